# baseline (speedup 1.0000x reference)
import functools

import jax
import jax.numpy as jnp
from jax import lax
from jax.experimental import pallas as pl
from jax.experimental.pallas import tpu as pltpu
from jax.experimental.pallas import tpu_sc as plsc

N = 10000
E = 320000
H = 128
C = 40
BN_EPS = 1e-5

NC = 2
NS = 16
NW = NC * NS
CHUNK = 128
GROUP = 8
CPW = 80
GPW = CPW // GROUP
E_PAD = NW * CPW * CHUNK
N_PAD = 10112
DUMMY_ROW = N_PAD - 1
ROWS_PER_TILE = N_PAD // NS


def _agg_body(x_hbm, src_hbm, dst_hbm, zeros_hbm, out_hbm,
              agg_sh, sidx, didx, rows_b, sem_i, sem_g):
    cid = lax.axis_index("c")
    sid = lax.axis_index("s")
    wid = sid * NC + cid
    base = wid * CPW

    pltpu.sync_copy(zeros_hbm.at[pl.ds(sid * ROWS_PER_TILE, ROWS_PER_TILE)],
                    agg_sh.at[pl.ds(sid * ROWS_PER_TILE, ROWS_PER_TILE)])

    def issue_idx(g, G):
        off = base + g * GROUP
        pltpu.async_copy(src_hbm.at[pl.ds(off, GROUP)], sidx[G], sem_i[G])
        pltpu.async_copy(dst_hbm.at[pl.ds(off, GROUP)], didx[G], sem_i[G])

    def wait_idx(g, G):
        off = base + g * GROUP
        pltpu.make_async_copy(src_hbm.at[pl.ds(off, GROUP)], sidx[G], sem_i[G]).wait()
        pltpu.make_async_copy(dst_hbm.at[pl.ds(off, GROUP)], didx[G], sem_i[G]).wait()

    def start_gather(G, j, b):
        pltpu.async_copy(x_hbm.at[sidx[G].at[j]], rows_b[b], sem_g[b])

    def wait_gather(G, j, b):
        pltpu.make_async_copy(x_hbm.at[sidx[G].at[j]], rows_b[b], sem_g[b]).wait()

    def scatter(G, j, b):
        pltpu.sync_copy(rows_b[b], agg_sh.at[didx[G].at[j]], add=True)

    def group_body(g, G, last, issue_next):
        for j in range(GROUP):
            b = j % 2
            if j < GROUP - 1:
                start_gather(G, j + 1, 1 - b)
            elif not last:
                wait_idx(g + 1, 1 - G)
                start_gather(1 - G, 0, 1 - b)
            wait_gather(G, j, b)
            scatter(G, j, b)
        if issue_next:
            issue_idx(g + 2, G)

    issue_idx(0, 0)
    issue_idx(1, 1)
    plsc.subcore_barrier()
    wait_idx(0, 0)
    start_gather(0, 0, 0)

    def pair_body(i, carry):
        group_body(2 * i, 0, last=False, issue_next=True)
        group_body(2 * i + 1, 1, last=False, issue_next=True)
        return carry

    lax.fori_loop(0, GPW // 2 - 1, pair_body, 0)
    group_body(GPW - 2, 0, last=False, issue_next=False)
    group_body(GPW - 1, 1, last=True, issue_next=False)

    plsc.subcore_barrier()
    pltpu.sync_copy(agg_sh.at[pl.ds(sid * ROWS_PER_TILE, ROWS_PER_TILE)],
                    out_hbm.at[cid, pl.ds(sid * ROWS_PER_TILE, ROWS_PER_TILE)])


@functools.cache
def _make_agg():
    return pl.kernel(
        _agg_body,
        out_type=jax.ShapeDtypeStruct((NC, N_PAD, H), jnp.float32),
        mesh=plsc.VectorSubcoreMesh(core_axis_name="c", subcore_axis_name="s"),
        scratch_types=[
            pltpu.VMEM_SHARED((N_PAD, H), jnp.float32),
            [pltpu.VMEM((GROUP, CHUNK), jnp.int32)] * 2,
            [pltpu.VMEM((GROUP, CHUNK), jnp.int32)] * 2,
            [pltpu.VMEM((CHUNK, H), jnp.float32)] * 2,
            [pltpu.SemaphoreType.DMA] * 2,
            [pltpu.SemaphoreType.DMA] * 2,
        ],
    )


def _agg(x, src2, dst2, zeros):
    p = _make_agg()(x, src2, dst2, zeros)
    return p[:, :N]


def _mlp_bn_body(eps_ref, x_ref, p0_ref, p1_ref, Wa_ref, ba_ref, Wb_ref,
                 bb_ref, g_ref, beta_ref, out_ref):
    h = x_ref[...] * (1.0 + eps_ref[0]) + (p0_ref[...] + p1_ref[...])
    h = jnp.maximum(jnp.dot(h, Wa_ref[...], preferred_element_type=jnp.float32)
                    + ba_ref[...], 0.0)
    h = jnp.maximum(jnp.dot(h, Wb_ref[...], preferred_element_type=jnp.float32)
                    + bb_ref[...], 0.0)
    mean = jnp.mean(h, axis=0, keepdims=True)
    var = jnp.mean(jnp.square(h - mean), axis=0, keepdims=True)
    out_ref[...] = (h - mean) * lax.rsqrt(var + BN_EPS) * g_ref[...] + beta_ref[...]


def _head_body(eps_ref, x_ref, p0_ref, p1_ref, Wa_ref, ba_ref, Wb_ref,
               bb_ref, g_ref, beta_ref, Wl1_ref, bl1_ref, Wl2_ref, bl2_ref,
               out_ref):
    h = x_ref[...] * (1.0 + eps_ref[0]) + (p0_ref[...] + p1_ref[...])
    h = jnp.maximum(jnp.dot(h, Wa_ref[...], preferred_element_type=jnp.float32)
                    + ba_ref[...], 0.0)
    h = jnp.maximum(jnp.dot(h, Wb_ref[...], preferred_element_type=jnp.float32)
                    + bb_ref[...], 0.0)
    mean = jnp.mean(h, axis=0, keepdims=True)
    var = jnp.mean(jnp.square(h - mean), axis=0, keepdims=True)
    h = (h - mean) * lax.rsqrt(var + BN_EPS) * g_ref[...] + beta_ref[...]
    h = jnp.maximum(jnp.dot(h, Wl1_ref[...], preferred_element_type=jnp.float32)
                    + bl1_ref[...], 0.0)
    out_ref[...] = (jnp.dot(h, Wl2_ref[...], preferred_element_type=jnp.float32)
                    + bl2_ref[...])


_SMEM1 = pl.BlockSpec(memory_space=pltpu.SMEM)


def _mlp_bn(eps, x, p0, p1, Wa, ba, Wb, bb, g, beta):
    return pl.pallas_call(
        _mlp_bn_body,
        out_shape=jax.ShapeDtypeStruct((N, H), jnp.float32),
        in_specs=[_SMEM1] + [pl.BlockSpec()] * 9,
        out_specs=pl.BlockSpec(),
    )(eps.reshape(1), x, p0, p1, Wa, ba, Wb, bb, g, beta)


def _head(eps, x, p0, p1, Wa, ba, Wb, bb, g, beta, Wl1, bl1, Wl2, bl2):
    return pl.pallas_call(
        _head_body,
        out_shape=jax.ShapeDtypeStruct((N, C), jnp.float32),
        in_specs=[_SMEM1] + [pl.BlockSpec()] * 13,
        out_specs=pl.BlockSpec(),
    )(eps.reshape(1), x, p0, p1, Wa, ba, Wb, bb, g, beta, Wl1, bl1, Wl2, bl2)


def kernel(x, edge_index, eps0, W0a, b0a, W0b, b0b, g0, beta0,
           eps1, W1a, b1a, W1b, b1b, g1, beta1,
           eps2, W2a, b2a, W2b, b2b, g2, beta2, Wl1, bl1, Wl2, bl2):
    npad = E_PAD - E
    src2 = jnp.concatenate(
        [edge_index[0], jnp.zeros((npad,), jnp.int32)]).reshape(-1, CHUNK)
    dst2 = jnp.concatenate(
        [edge_index[1], jnp.full((npad,), DUMMY_ROW, jnp.int32)]).reshape(-1, CHUNK)
    zeros = jnp.zeros((N_PAD, H), jnp.float32)

    p = _agg(x, src2, dst2, zeros)
    h = _mlp_bn(eps0, x, p[0], p[1], W0a, b0a, W0b, b0b, g0, beta0)
    p = _agg(h, src2, dst2, zeros)
    h = _mlp_bn(eps1, h, p[0], p[1], W1a, b1a, W1b, b1b, g1, beta1)
    p = _agg(h, src2, dst2, zeros)
    return _head(eps2, h, p[0], p[1], W2a, b2a, W2b, b2b, g2, beta2,
                 Wl1, bl1, Wl2, bl2)


# speedup vs baseline: 3.4421x; 3.4421x over previous
import functools

import jax
import jax.numpy as jnp
from jax import lax
from jax.experimental import pallas as pl
from jax.experimental.pallas import tpu as pltpu
from jax.experimental.pallas import tpu_sc as plsc

N = 10000
E = 320000
H = 128
C = 40
BN_EPS = 1e-5

NC = 2
NS = 16
NW = NC * NS
EPW = E // NW
CHUNK = 96
FULL_CHUNKS = EPW // CHUNK
REM = EPW - FULL_CHUNKS * CHUNK
N_PAD = 10112
ROWS_PER_TILE = N_PAD // NS


def _agg_body(x_hbm, src_hbm, dst_hbm, zeros_hbm, out_hbm,
              agg_sh, src_t, dst_t, rows_t, srcr_v, dstr_v, rowsr_v,
              sem_g, sem_i, sem_r):
    cid = lax.axis_index("c")
    sid = lax.axis_index("s")
    wid = sid * NC + cid
    base = wid * EPW

    pltpu.sync_copy(zeros_hbm.at[pl.ds(sid * ROWS_PER_TILE, ROWS_PER_TILE)],
                    agg_sh.at[pl.ds(sid * ROWS_PER_TILE, ROWS_PER_TILE)])

    def issue_idx(c, q):
        off = base + c * CHUNK
        pltpu.async_copy(src_hbm.at[pl.ds(off, CHUNK)], src_t[q], sem_i[q])
        pltpu.async_copy(dst_hbm.at[pl.ds(off, CHUNK)], dst_t[q], sem_i[q])

    def wait_idx(c, q):
        off = base + c * CHUNK
        pltpu.make_async_copy(src_hbm.at[pl.ds(off, CHUNK)], src_t[q], sem_i[q]).wait()
        pltpu.make_async_copy(dst_hbm.at[pl.ds(off, CHUNK)], dst_t[q], sem_i[q]).wait()

    def start_gather(q):
        pltpu.async_copy(x_hbm.at[src_t[q]], rows_t[q], sem_g[q])

    def wait_gather(q):
        pltpu.make_async_copy(x_hbm.at[src_t[q]], rows_t[q], sem_g[q]).wait()

    def scatter(q):
        pltpu.sync_copy(rows_t[q], agg_sh.at[dst_t[q]], add=True)

    def step(c, q, g_ahead=True, i_ahead=True):
        if g_ahead:
            wait_idx(c + 2, (q + 2) % 3)
            start_gather((q + 2) % 3)
        wait_gather(q)
        scatter(q)
        if i_ahead:
            issue_idx(c + 3, q)

    issue_idx(0, 0)
    issue_idx(1, 1)
    issue_idx(2, 2)
    plsc.subcore_barrier()
    wait_idx(0, 0)
    start_gather(0)
    wait_idx(1, 1)
    start_gather(1)

    def tri_step(i, carry):
        c = 3 * i
        step(c, 0)
        step(c + 1, 1)
        step(c + 2, 2)
        return carry

    lax.fori_loop(0, FULL_CHUNKS // 3 - 1, tri_step, 0)

    step(FULL_CHUNKS - 5, (FULL_CHUNKS - 5) % 3)
    step(FULL_CHUNKS - 4, (FULL_CHUNKS - 4) % 3)
    step(FULL_CHUNKS - 3, (FULL_CHUNKS - 3) % 3, i_ahead=False)
    step(FULL_CHUNKS - 2, (FULL_CHUNKS - 2) % 3, g_ahead=False, i_ahead=False)
    step(FULL_CHUNKS - 1, (FULL_CHUNKS - 1) % 3, g_ahead=False, i_ahead=False)

    off = base + FULL_CHUNKS * CHUNK
    pltpu.sync_copy(src_hbm.at[pl.ds(off, REM)], srcr_v)
    pltpu.sync_copy(dst_hbm.at[pl.ds(off, REM)], dstr_v)
    pltpu.async_copy(x_hbm.at[srcr_v], rowsr_v, sem_r).wait()
    pltpu.sync_copy(rowsr_v, agg_sh.at[dstr_v], add=True)

    plsc.subcore_barrier()
    pltpu.sync_copy(agg_sh.at[pl.ds(sid * ROWS_PER_TILE, ROWS_PER_TILE)],
                    out_hbm.at[cid, pl.ds(sid * ROWS_PER_TILE, ROWS_PER_TILE)])


@functools.cache
def _make_agg():
    return pl.kernel(
        _agg_body,
        out_type=jax.ShapeDtypeStruct((NC, N_PAD, H), jnp.float32),
        mesh=plsc.VectorSubcoreMesh(core_axis_name="c", subcore_axis_name="s"),
        scratch_types=[
            pltpu.VMEM_SHARED((N_PAD, H), jnp.float32),
            [pltpu.VMEM((CHUNK,), jnp.int32)] * 3,
            [pltpu.VMEM((CHUNK,), jnp.int32)] * 3,
            [pltpu.VMEM((CHUNK, H), jnp.float32)] * 3,
            pltpu.VMEM((REM,), jnp.int32),
            pltpu.VMEM((REM,), jnp.int32),
            pltpu.VMEM((REM, H), jnp.float32),
            [pltpu.SemaphoreType.DMA] * 3,
            [pltpu.SemaphoreType.DMA] * 3,
            pltpu.SemaphoreType.DMA,
        ],
    )


def _agg(x, src, dst, zeros):
    p = _make_agg()(x, src, dst, zeros)
    return p[:, :N]


def _mlp_bn_body(eps_ref, x_ref, p0_ref, p1_ref, Wa_ref, ba_ref, Wb_ref,
                 bb_ref, g_ref, beta_ref, out_ref):
    h = x_ref[...] * (1.0 + eps_ref[0]) + (p0_ref[...] + p1_ref[...])
    h = jnp.maximum(jnp.dot(h, Wa_ref[...], preferred_element_type=jnp.float32)
                    + ba_ref[...], 0.0)
    h = jnp.maximum(jnp.dot(h, Wb_ref[...], preferred_element_type=jnp.float32)
                    + bb_ref[...], 0.0)
    mean = jnp.mean(h, axis=0, keepdims=True)
    var = jnp.mean(jnp.square(h - mean), axis=0, keepdims=True)
    out_ref[...] = (h - mean) * lax.rsqrt(var + BN_EPS) * g_ref[...] + beta_ref[...]


def _head_body(eps_ref, x_ref, p0_ref, p1_ref, Wa_ref, ba_ref, Wb_ref,
               bb_ref, g_ref, beta_ref, Wl1_ref, bl1_ref, Wl2_ref, bl2_ref,
               out_ref):
    h = x_ref[...] * (1.0 + eps_ref[0]) + (p0_ref[...] + p1_ref[...])
    h = jnp.maximum(jnp.dot(h, Wa_ref[...], preferred_element_type=jnp.float32)
                    + ba_ref[...], 0.0)
    h = jnp.maximum(jnp.dot(h, Wb_ref[...], preferred_element_type=jnp.float32)
                    + bb_ref[...], 0.0)
    mean = jnp.mean(h, axis=0, keepdims=True)
    var = jnp.mean(jnp.square(h - mean), axis=0, keepdims=True)
    h = (h - mean) * lax.rsqrt(var + BN_EPS) * g_ref[...] + beta_ref[...]
    h = jnp.maximum(jnp.dot(h, Wl1_ref[...], preferred_element_type=jnp.float32)
                    + bl1_ref[...], 0.0)
    out_ref[...] = (jnp.dot(h, Wl2_ref[...], preferred_element_type=jnp.float32)
                    + bl2_ref[...])


_SMEM1 = pl.BlockSpec(memory_space=pltpu.SMEM)


def _mlp_bn(eps, x, p0, p1, Wa, ba, Wb, bb, g, beta):
    return pl.pallas_call(
        _mlp_bn_body,
        out_shape=jax.ShapeDtypeStruct((N, H), jnp.float32),
        in_specs=[_SMEM1] + [pl.BlockSpec()] * 9,
        out_specs=pl.BlockSpec(),
    )(eps.reshape(1), x, p0, p1, Wa, ba, Wb, bb, g, beta)


def _head(eps, x, p0, p1, Wa, ba, Wb, bb, g, beta, Wl1, bl1, Wl2, bl2):
    return pl.pallas_call(
        _head_body,
        out_shape=jax.ShapeDtypeStruct((N, C), jnp.float32),
        in_specs=[_SMEM1] + [pl.BlockSpec()] * 13,
        out_specs=pl.BlockSpec(),
    )(eps.reshape(1), x, p0, p1, Wa, ba, Wb, bb, g, beta, Wl1, bl1, Wl2, bl2)


def kernel(x, edge_index, eps0, W0a, b0a, W0b, b0b, g0, beta0,
           eps1, W1a, b1a, W1b, b1b, g1, beta1,
           eps2, W2a, b2a, W2b, b2b, g2, beta2, Wl1, bl1, Wl2, bl2):
    src = edge_index[0]
    dst = edge_index[1]
    zeros = jnp.zeros((N_PAD, H), jnp.float32)

    p = _agg(x, src, dst, zeros)
    h = _mlp_bn(eps0, x, p[0], p[1], W0a, b0a, W0b, b0b, g0, beta0)
    p = _agg(h, src, dst, zeros)
    h = _mlp_bn(eps1, h, p[0], p[1], W1a, b1a, W1b, b1b, g1, beta1)
    p = _agg(h, src, dst, zeros)
    return _head(eps2, h, p[0], p[1], W2a, b2a, W2b, b2b, g2, beta2,
                 Wl1, bl1, Wl2, bl2)
